# Initial kernel scaffold; baseline (speedup 1.0000x reference)
import functools

import jax
import jax.numpy as jnp
from jax import lax
from jax.experimental import pallas as pl
from jax.experimental.pallas import tpu as pltpu
from jax.experimental.pallas import tpu_sc as plsc

VOCAB = 100000
EMB = 64
EPS = 1e-5

NC = 2
NS = 16
NW = NC * NS

ROW_BLOCK = 5000

CH = 128


def _ln_table_body(w_ref, g_ref, b_ref, o_ref):
    x = w_ref[...]
    mean = jnp.mean(x, axis=-1, keepdims=True)
    xc = x - mean
    var = jnp.mean(xc * xc, axis=-1, keepdims=True)
    o_ref[...] = xc * lax.rsqrt(var + EPS) * g_ref[...] + b_ref[...]


def _normalize_table(table, gamma, beta):
    grid = VOCAB // ROW_BLOCK
    return pl.pallas_call(
        _ln_table_body,
        grid=(grid,),
        in_specs=[
            pl.BlockSpec((ROW_BLOCK, EMB), lambda i: (i, 0)),
            pl.BlockSpec((1, EMB), lambda i: (0, 0)),
            pl.BlockSpec((1, EMB), lambda i: (0, 0)),
        ],
        out_specs=pl.BlockSpec((ROW_BLOCK, EMB), lambda i: (i, 0)),
        out_shape=jax.ShapeDtypeStruct((VOCAB, EMB), jnp.float32),
    )(table, gamma.reshape(1, EMB), beta.reshape(1, EMB))


def _make_gather(B):
    b_per_w = B // NW
    nch = b_per_w // CH
    mesh = plsc.VectorSubcoreMesh(core_axis_name="c", subcore_axis_name="s")

    @functools.partial(
        pl.kernel,
        mesh=mesh,
        out_type=jax.ShapeDtypeStruct((B, EMB), jnp.float32),
        scratch_types=[
            pltpu.VMEM((b_per_w,), jnp.int32),
            pltpu.VMEM((2, CH, EMB), jnp.float32),
            pltpu.SemaphoreType.DMA,
            pltpu.SemaphoreType.DMA,
        ],
    )
    def gather_kernel(table_hbm, idx_hbm, out_hbm, idx_v, rows_v, sem0, sem1):
        sems = (sem0, sem1)
        wid = lax.axis_index("s") * NC + lax.axis_index("c")
        base = wid * b_per_w
        pltpu.sync_copy(idx_hbm.at[pl.ds(base, b_per_w)], idx_v)
        pltpu.async_copy(
            table_hbm.at[idx_v.at[pl.ds(0, CH)]], rows_v.at[0], sem0)
        pltpu.async_copy(
            table_hbm.at[idx_v.at[pl.ds(CH, CH)]], rows_v.at[1], sem1)

        def body(i, _):
            j0 = i * 2
            for b in range(2):
                j = j0 + b
                pltpu.make_async_copy(
                    table_hbm.at[pl.ds(0, CH)], rows_v.at[b], sems[b]).wait()
                pltpu.sync_copy(
                    rows_v.at[b], out_hbm.at[pl.ds(base + j * CH, CH)])

                @pl.when(j + 2 < nch)
                def _():
                    pltpu.async_copy(
                        table_hbm.at[idx_v.at[pl.ds((j + 2) * CH, CH)]],
                        rows_v.at[b], sems[b])
            return 0

        lax.fori_loop(0, nch // 2, body, 0)

    return gather_kernel


def kernel(input_tensor, token_emb_weight, ln_weight, ln_bias):
    batch, seq = input_tensor.shape
    B = batch * seq
    normed = _normalize_table(token_emb_weight, ln_weight, ln_bias)
    flat_idx = input_tensor.reshape(B)
    out = _make_gather(B)(normed, flat_idx)
    return out.reshape(batch, seq, EMB)


# speedup vs baseline: 5.5079x; 5.5079x over previous
import functools

import jax
import jax.numpy as jnp
from jax import lax
from jax.experimental import pallas as pl
from jax.experimental.pallas import tpu as pltpu
from jax.experimental.pallas import tpu_sc as plsc

VOCAB = 100000
EMB = 64
EPS = 1e-5

NC = 2
NS = 16
NW = NC * NS

ROW_BLOCK = 5000

CH = 128


def _ln_table_body(w_ref, g_ref, b_ref, o_ref):
    x = w_ref[...]
    mean = jnp.mean(x, axis=-1, keepdims=True)
    xc = x - mean
    var = jnp.mean(xc * xc, axis=-1, keepdims=True)
    n = xc * lax.rsqrt(var + EPS) * g_ref[...] + b_ref[...]
    o_ref[...] = jnp.concatenate([n, jnp.zeros_like(n)], axis=-1)


def _normalize_table(table, gamma, beta):
    grid = VOCAB // ROW_BLOCK
    return pl.pallas_call(
        _ln_table_body,
        grid=(grid,),
        in_specs=[
            pl.BlockSpec((ROW_BLOCK, EMB), lambda i: (i, 0)),
            pl.BlockSpec((1, EMB), lambda i: (0, 0)),
            pl.BlockSpec((1, EMB), lambda i: (0, 0)),
        ],
        out_specs=pl.BlockSpec((ROW_BLOCK, 2 * EMB), lambda i: (i, 0)),
        out_shape=jax.ShapeDtypeStruct((VOCAB, 2 * EMB), jnp.float32),
    )(table, gamma.reshape(1, EMB), beta.reshape(1, EMB))


def _make_gather(B):
    b_per_w = B // NW
    nch = b_per_w // CH
    mesh = plsc.VectorSubcoreMesh(core_axis_name="c", subcore_axis_name="s")

    @functools.partial(
        pl.kernel,
        mesh=mesh,
        out_type=jax.ShapeDtypeStruct((B, EMB), jnp.float32),
        scratch_types=[
            pltpu.VMEM((b_per_w,), jnp.int32),
            pltpu.VMEM((2, CH, 2 * EMB), jnp.float32),
            pltpu.VMEM((2, CH, EMB), jnp.float32),
            pltpu.SemaphoreType.DMA,
            pltpu.SemaphoreType.DMA,
            pltpu.SemaphoreType.DMA,
            pltpu.SemaphoreType.DMA,
        ],
    )
    def gather_kernel(table_hbm, idx_hbm, out_hbm, idx_v, rows_v, comp_v,
                      gsem0, gsem1, wsem0, wsem1):
        gsems = (gsem0, gsem1)
        wsems = (wsem0, wsem1)
        wid = lax.axis_index("s") * NC + lax.axis_index("c")
        base = wid * b_per_w
        pltpu.sync_copy(idx_hbm.at[pl.ds(base, b_per_w)], idx_v)
        pltpu.async_copy(
            table_hbm.at[idx_v.at[pl.ds(0, CH)]], rows_v.at[0], gsem0)
        pltpu.async_copy(
            table_hbm.at[idx_v.at[pl.ds(CH, CH)]], rows_v.at[1], gsem1)

        def compact(b):
            def row(r, _):
                for c in range(EMB // 16):
                    comp_v[b, r, pl.ds(c * 16, 16)] = (
                        rows_v[b, r, pl.ds(c * 16, 16)])
                return 0
            lax.fori_loop(0, CH, row, 0)

        def body(i, _):
            j0 = i * 2
            for b in range(2):
                j = j0 + b
                pltpu.make_async_copy(
                    table_hbm.at[pl.ds(0, CH)], rows_v.at[b], gsems[b]).wait()

                @pl.when(j >= 2)
                def _():
                    pltpu.make_async_copy(
                        comp_v.at[b],
                        out_hbm.at[pl.ds(0, CH)], wsems[b]).wait()

                compact(b)
                pltpu.async_copy(
                    comp_v.at[b], out_hbm.at[pl.ds(base + j * CH, CH)],
                    wsems[b])

                @pl.when(j + 2 < nch)
                def _():
                    pltpu.async_copy(
                        table_hbm.at[idx_v.at[pl.ds((j + 2) * CH, CH)]],
                        rows_v.at[b], gsems[b])
            return 0

        lax.fori_loop(0, nch // 2, body, 0)
        for b in range(2):
            pltpu.make_async_copy(
                comp_v.at[b], out_hbm.at[pl.ds(0, CH)], wsems[b]).wait()

    return gather_kernel


def kernel(input_tensor, token_emb_weight, ln_weight, ln_bias):
    batch, seq = input_tensor.shape
    B = batch * seq
    normed = _normalize_table(token_emb_weight, ln_weight, ln_bias)
    flat_idx = input_tensor.reshape(B)
    out = _make_gather(B)(normed, flat_idx)
    return out.reshape(batch, seq, EMB)
